# baseline (speedup 1.0000x reference)
import jax
import jax.numpy as jnp
from jax.experimental import pallas as pl
from jax.experimental.pallas import tpu as pltpu

_ALPHA = 4
_FAST_SPLIT = 8


def kernel(frames):
    B, T, C, H, W = frames.shape
    S = T // _ALPHA
    F = C * H * W
    N = B * T * F

    def body(src, slow, fast, sem):
        cps = []
        span = N // _FAST_SPLIT
        for i in range(_FAST_SPLIT):
            cps.append(pltpu.make_async_copy(
                src.at[pl.ds(i * span, span)],
                fast.at[pl.ds(i * span, span)], sem))
        for w in range(B * S):
            b, j = w // S, w % S
            srow = b * T + (j * (T - 1)) // (S - 1)
            cps.append(pltpu.make_async_copy(
                src.at[pl.ds(srow * F, F)],
                slow.at[pl.ds(w * F, F)], sem))
        for c in cps:
            c.start()
        for c in cps:
            c.wait()

    flat = frames.reshape(N)
    slow, fast = pl.pallas_call(
        body,
        in_specs=[pl.BlockSpec(memory_space=pl.ANY)],
        out_specs=[
            pl.BlockSpec(memory_space=pl.ANY),
            pl.BlockSpec(memory_space=pl.ANY),
        ],
        out_shape=[
            jax.ShapeDtypeStruct((B * S * F,), frames.dtype),
            jax.ShapeDtypeStruct((B * T * F,), frames.dtype),
        ],
        scratch_shapes=[pltpu.SemaphoreType.DMA],
    )(flat)
    return slow.reshape(B, S, C, H, W), fast.reshape(B, T, C, H, W)


# speedup vs baseline: 11.1928x; 11.1928x over previous
import jax
import jax.numpy as jnp
from jax.experimental import pallas as pl
from jax.experimental.pallas import tpu as pltpu

_ALPHA = 4
_K = 16
_L = 8


def kernel(frames):
    B, T, C, H, W = frames.shape
    S = T // _ALPHA
    F = C * H * W
    sel = {b * T + (j * (T - 1)) // (S - 1): b * S + j
           for b in range(B) for j in range(S)}
    n = B * T

    def body(src, slow, fast, buf, *sems):
        isem, osem = sems[:_K], sems[_K:]
        ins, outs = [None] * n, [None] * n

        def start_in(i):
            s = i % _K
            ins[i] = pltpu.make_async_copy(
                src.at[pl.ds(i * F, F)], buf.at[pl.ds(s * F, F)], isem[s])
            ins[i].start()

        def start_outs(i):
            s = i % _K
            cs = [pltpu.make_async_copy(
                buf.at[pl.ds(s * F, F)], fast.at[pl.ds(i * F, F)], osem[s])]
            if i in sel:
                cs.append(pltpu.make_async_copy(
                    buf.at[pl.ds(s * F, F)],
                    slow.at[pl.ds(sel[i] * F, F)], osem[s]))
            for c in cs:
                c.start()
            outs[i] = cs

        for i in range(_L):
            start_in(i)
        for i in range(n):
            ins[i].wait()
            start_outs(i)
            j = i + _L
            if j < n:
                if j - _K >= 0:
                    for oc in outs[j - _K]:
                        oc.wait()
                start_in(j)
        for i in range(n - _K, n):
            for oc in outs[i]:
                oc.wait()

    flat = frames.reshape(B * T * F)
    slow, fast = pl.pallas_call(
        body,
        in_specs=[pl.BlockSpec(memory_space=pl.ANY)],
        out_specs=[
            pl.BlockSpec(memory_space=pl.ANY),
            pl.BlockSpec(memory_space=pl.ANY),
        ],
        out_shape=[
            jax.ShapeDtypeStruct((B * S * F,), frames.dtype),
            jax.ShapeDtypeStruct((B * T * F,), frames.dtype),
        ],
        scratch_shapes=[pltpu.VMEM((_K * F,), frames.dtype)]
        + [pltpu.SemaphoreType.DMA] * (2 * _K),
    )(flat)
    return slow.reshape(B, S, C, H, W), fast.reshape(B, T, C, H, W)


# speedup vs baseline: 14.3557x; 1.2826x over previous
import functools

import jax
import jax.numpy as jnp
from jax import lax
from jax.experimental import pallas as pl
from jax.experimental.pallas import tpu as pltpu
from jax.experimental.pallas import tpu_sc as plsc

_ALPHA = 4
_SPLIT = 4
_NBUF = 3


def _make_sc_gather(B, T, F, dtype):
    S = T // _ALPHA
    CH = F // _SPLIT
    n_jobs = _SPLIT
    mesh = plsc.VectorSubcoreMesh(core_axis_name="c", subcore_axis_name="s")

    @functools.partial(
        pl.kernel,
        out_type=jax.ShapeDtypeStruct((B * S * F,), dtype),
        mesh=mesh,
        scratch_types=[pltpu.VMEM((_NBUF * CH,), dtype)]
        + [pltpu.SemaphoreType.DMA] * (2 * _NBUF),
    )
    def k(src_hbm, slow_hbm, buf, *sems):
        isem, osem = sems[:_NBUF], sems[_NBUF:]
        wid = lax.axis_index("s") * 2 + lax.axis_index("c")
        b = wid // S
        j = wid % S
        t_src = (j * (T - 1)) // (S - 1)
        src_base = (b * T + t_src) * F
        dst_base = wid * F

        ins, outs = [None] * n_jobs, [None] * n_jobs

        def start_in(i):
            ins[i] = pltpu.make_async_copy(
                src_hbm.at[pl.ds(src_base + i * CH, CH)],
                buf.at[pl.ds((i % _NBUF) * CH, CH)], isem[i % _NBUF])
            ins[i].start()

        def start_out(i):
            outs[i] = pltpu.make_async_copy(
                buf.at[pl.ds((i % _NBUF) * CH, CH)],
                slow_hbm.at[pl.ds(dst_base + i * CH, CH)], osem[i % _NBUF])
            outs[i].start()

        start_in(0)
        for i in range(n_jobs):
            if i + 1 < n_jobs:
                if i + 1 >= _NBUF:
                    outs[i + 1 - _NBUF].wait()
                start_in(i + 1)
            ins[i].wait()
            start_out(i)
        for i in range(max(0, n_jobs - _NBUF), n_jobs):
            outs[i].wait()

    return k


def kernel(frames):
    B, T, C, H, W = frames.shape
    S = T // _ALPHA
    F = C * H * W
    flat = frames.reshape(B * T * F)
    slow = _make_sc_gather(B, T, F, frames.dtype)(flat)
    return slow.reshape(B, S, C, H, W), frames


# speedup vs baseline: 37.7686x; 2.6309x over previous
import jax
import jax.numpy as jnp
from jax.experimental import pallas as pl

_ALPHA = 4


def kernel(frames):
    B, T, C, H, W = frames.shape
    S = T // _ALPHA

    def in_map(b, j):
        return (b, (j * (T - 1)) // (S - 1), 0, 0, 0)

    def body(in_ref, out_ref):
        out_ref[...] = in_ref[...]

    slow = pl.pallas_call(
        body,
        grid=(B, S),
        in_specs=[pl.BlockSpec((1, 1, C, H, W), in_map)],
        out_specs=pl.BlockSpec((1, 1, C, H, W), lambda b, j: (b, j, 0, 0, 0)),
        out_shape=jax.ShapeDtypeStruct((B, S, C, H, W), frames.dtype),
    )(frames)
    return slow, frames
